# baseline (speedup 1.0000x reference)
import functools

import jax
import jax.numpy as jnp
from jax import lax
from jax.experimental import pallas as pl
from jax.experimental.pallas import tpu as pltpu
from jax.experimental.pallas import tpu_sc as plsc

N = 10000
E = 320000
D = 128

NC = 2
NS = 16
NW = NC * NS
C = 128
CPT = 80
EP = NW * CPT * C
EPT = CPT * C
NP = N + 48
RPT = 624
TAIL = N - NS * RPT

_mesh = plsc.VectorSubcoreMesh(core_axis_name="c", subcore_axis_name="s")


@functools.partial(
    pl.kernel,
    out_type=jax.ShapeDtypeStruct((NC * N, D), jnp.float32),
    mesh=_mesh,
    scratch_types=(
        [pltpu.VMEM((CPT, C), jnp.int32)]
        + [pltpu.VMEM((C, D), jnp.float32)]
        + [pltpu.SemaphoreType.DMA] * 4
        + [pltpu.VMEM_SHARED((NP, D), jnp.float32)]
    ),
)
def _deg_sc(dst2d_hbm, ones_hbm, zeros_hbm, out_hbm,
            didx, ones_v, s0s, s1s, s2s, s3s, acc):
    c = lax.axis_index("c")
    s = lax.axis_index("s")
    wid = c * NS + s
    ssem = (s0s, s1s, s2s, s3s)

    pltpu.sync_copy(zeros_hbm.at[pl.ds(s * RPT, RPT)], acc.at[pl.ds(s * RPT, RPT)])
    @pl.when(s == 0)
    def _():
        pltpu.sync_copy(zeros_hbm.at[pl.ds(NS * RPT, TAIL)],
                        acc.at[pl.ds(NS * RPT, TAIL)])
    pltpu.sync_copy(dst2d_hbm.at[pl.ds(wid * CPT, CPT)], didx)
    pltpu.sync_copy(ones_hbm, ones_v)
    plsc.subcore_barrier()

    def issue(t, b):
        pltpu.async_copy(ones_v, acc.at[didx.at[t]], ssem[b], add=True)

    def wait(t, b):
        pltpu.make_async_copy(ones_v, acc.at[didx.at[t]], ssem[b]).wait()

    issue(0, 0)
    issue(1, 1)
    issue(2, 2)
    issue(3, 3)

    def body(p, carry):
        t = 4 * p
        wait(t - 4, 0); issue(t, 0)
        wait(t - 3, 1); issue(t + 1, 1)
        wait(t - 2, 2); issue(t + 2, 2)
        wait(t - 1, 3); issue(t + 3, 3)
        return carry

    lax.fori_loop(1, 20, body, 0)
    wait(76, 0)
    wait(77, 1)
    wait(78, 2)
    wait(79, 3)

    plsc.subcore_barrier()
    pltpu.sync_copy(acc.at[pl.ds(s * RPT, RPT)],
                    out_hbm.at[pl.ds(c * N + s * RPT, RPT)])
    @pl.when(s == 0)
    def _():
        pltpu.sync_copy(acc.at[pl.ds(NS * RPT, TAIL)],
                        out_hbm.at[pl.ds(c * N + NS * RPT, TAIL)])


@functools.partial(
    pl.kernel,
    out_type=jax.ShapeDtypeStruct((NC * N, D), jnp.float32),
    mesh=_mesh,
    scratch_types=(
        [pltpu.VMEM((CPT, C), jnp.int32)]
        + [pltpu.VMEM((C,), jnp.int32)] * 2
        + [pltpu.VMEM((C, D), jnp.float32)] * 2
        + [pltpu.SemaphoreType.DMA] * 4
        + [pltpu.VMEM_SHARED((NP, D), jnp.float32)]
    ),
)
def _edge_sc(src_hbm, dst2d_hbm, g_hbm, zeros_hbm, out_hbm,
             didx, sb0, sb1, r0, r1, g0s, g1s, i0s, i1s, acc):
    c = lax.axis_index("c")
    s = lax.axis_index("s")
    wid = c * NS + s
    base = wid * EPT
    rows = (r0, r1)
    sbuf = (sb0, sb1)
    gsem = (g0s, g1s)
    isem = (i0s, i1s)

    pltpu.sync_copy(zeros_hbm.at[pl.ds(s * RPT, RPT)], acc.at[pl.ds(s * RPT, RPT)])
    @pl.when(s == 0)
    def _():
        pltpu.sync_copy(zeros_hbm.at[pl.ds(NS * RPT, TAIL)],
                        acc.at[pl.ds(NS * RPT, TAIL)])
    pltpu.sync_copy(dst2d_hbm.at[pl.ds(wid * CPT, CPT)], didx)
    plsc.subcore_barrier()

    def sidx_pre(t, b):
        return pltpu.async_copy(src_hbm.at[pl.ds(base + t * C, C)],
                                sbuf[b], isem[b])

    def g_wait(b):
        pltpu.make_async_copy(g_hbm.at[sbuf[b]], rows[b], gsem[b]).wait()

    sidx_pre(0, 0).wait()
    pltpu.async_copy(g_hbm.at[sbuf[0]], rows[0], gsem[0])
    sidx_pre(1, 1)

    def visit(t, b, do_g, do_pre):
        g_wait(b)
        if do_pre:
            sidx_pre(t + 2, b)
        if do_g:
            pltpu.make_async_copy(src_hbm.at[pl.ds(base + (t + 1) * C, C)],
                                  sbuf[1 - b], isem[1 - b]).wait()
            pltpu.async_copy(g_hbm.at[sbuf[1 - b]], rows[1 - b], gsem[1 - b])
        pltpu.sync_copy(rows[b], acc.at[didx.at[t]], add=True)

    def body(p, carry):
        t = 2 * p
        visit(t, 0, True, True)
        visit(t + 1, 1, True, True)
        return carry

    lax.fori_loop(0, 39, body, 0)
    visit(78, 0, True, False)
    visit(79, 1, False, False)

    plsc.subcore_barrier()
    pltpu.sync_copy(acc.at[pl.ds(s * RPT, RPT)],
                    out_hbm.at[pl.ds(c * N + s * RPT, RPT)])
    @pl.when(s == 0)
    def _():
        pltpu.sync_copy(acc.at[pl.ds(NS * RPT, TAIL)],
                        out_hbm.at[pl.ds(c * N + NS * RPT, TAIL)])


_R = 1000
_NB = N // _R


def _lin_tc_body(x_ref, win_ref, bin_ref, w1_ref, u_ref):
    h0 = jnp.dot(x_ref[...], win_ref[...],
                 preferred_element_type=jnp.float32) + bin_ref[...]
    u_ref[...] = jnp.dot(h0, w1_ref[...], preferred_element_type=jnp.float32)


def _lin_tc(x, w_in, b_in, w1):
    return pl.pallas_call(
        _lin_tc_body,
        grid=(_NB,),
        in_specs=[
            pl.BlockSpec((_R, D), lambda i: (i, 0)),
            pl.BlockSpec((D, D), lambda i: (0, 0)),
            pl.BlockSpec((1, D), lambda i: (0, 0)),
            pl.BlockSpec((D, D), lambda i: (0, 0)),
        ],
        out_specs=pl.BlockSpec((_R, D), lambda i: (i, 0)),
        out_shape=jax.ShapeDtypeStruct((N, D), jnp.float32),
    )(x, w_in, b_in, w1)


def _scale_tc_body(u_ref, ca_ref, cb_ref, dis_ref, g_ref):
    deg = 1.0 + ca_ref[:, 0:1] + cb_ref[:, 0:1]
    dis = lax.rsqrt(deg)
    dis_ref[...] = dis
    g_ref[...] = u_ref[...] * dis


def _scale_tc(u, cnt):
    return pl.pallas_call(
        _scale_tc_body,
        grid=(_NB,),
        in_specs=[
            pl.BlockSpec((_R, D), lambda i: (i, 0)),
            pl.BlockSpec((_R, D), lambda i: (i, 0)),
            pl.BlockSpec((_R, D), lambda i: (_NB + i, 0)),
        ],
        out_specs=[
            pl.BlockSpec((_R, 1), lambda i: (i, 0)),
            pl.BlockSpec((_R, D), lambda i: (i, 0)),
        ],
        out_shape=[
            jax.ShapeDtypeStruct((N, 1), jnp.float32),
            jax.ShapeDtypeStruct((N, D), jnp.float32),
        ],
    )(u, cnt, cnt)


def _mid_tc_body(sa_ref, sb_ref, g_ref, dis_ref, b_ref, w_ref, gn_ref):
    dis = dis_ref[...]
    out = dis * (sa_ref[...] + sb_ref[...] + g_ref[...]) + b_ref[...]
    gn_ref[...] = jnp.dot(out, w_ref[...],
                          preferred_element_type=jnp.float32) * dis


def _mid_tc(s_part, g, dis, b, w):
    return pl.pallas_call(
        _mid_tc_body,
        grid=(_NB,),
        in_specs=[
            pl.BlockSpec((_R, D), lambda i: (i, 0)),
            pl.BlockSpec((_R, D), lambda i: (_NB + i, 0)),
            pl.BlockSpec((_R, D), lambda i: (i, 0)),
            pl.BlockSpec((_R, 1), lambda i: (i, 0)),
            pl.BlockSpec((1, D), lambda i: (0, 0)),
            pl.BlockSpec((D, D), lambda i: (0, 0)),
        ],
        out_specs=pl.BlockSpec((_R, D), lambda i: (i, 0)),
        out_shape=jax.ShapeDtypeStruct((N, D), jnp.float32),
    )(s_part, s_part, g, dis, b, w)


def _final_tc_body(sa_ref, sb_ref, g_ref, dis_ref, b_ref, out_ref):
    out_ref[...] = dis_ref[...] * (sa_ref[...] + sb_ref[...] + g_ref[...]) \
        + b_ref[...]


def _final_tc(s_part, g, dis, b):
    return pl.pallas_call(
        _final_tc_body,
        grid=(_NB,),
        in_specs=[
            pl.BlockSpec((_R, D), lambda i: (i, 0)),
            pl.BlockSpec((_R, D), lambda i: (_NB + i, 0)),
            pl.BlockSpec((_R, D), lambda i: (i, 0)),
            pl.BlockSpec((_R, 1), lambda i: (i, 0)),
            pl.BlockSpec((1, D), lambda i: (0, 0)),
        ],
        out_specs=pl.BlockSpec((_R, D), lambda i: (i, 0)),
        out_shape=jax.ShapeDtypeStruct((N, D), jnp.float32),
    )(s_part, s_part, g, dis, b)


def kernel(x, edge_index, W_in, b_in, W1, b1, W2, b2):
    pad = EP - E
    src_p = jnp.concatenate([edge_index[0], jnp.zeros((pad,), jnp.int32)])
    pad_dst = N + (jnp.arange(pad, dtype=jnp.int32) % (NP - N))
    dst2d = jnp.concatenate([edge_index[1], pad_dst]).reshape(-1, C)
    zeros = jnp.zeros((N, D), jnp.float32)
    ones = jnp.ones((C, D), jnp.float32)

    cnt = _deg_sc(dst2d, ones, zeros)
    u = _lin_tc(x, W_in, b_in.reshape(1, D), W1)
    dis, g0 = _scale_tc(u, cnt)

    s0 = _edge_sc(src_p, dst2d, g0, zeros)
    g1 = _mid_tc(s0, g0, dis, b1.reshape(1, D), W2)

    s1 = _edge_sc(src_p, dst2d, g1, zeros)
    return _final_tc(s1, g1, dis, b2.reshape(1, D))


# speedup vs baseline: 1.2029x; 1.2029x over previous
import functools

import jax
import jax.numpy as jnp
from jax import lax
from jax.experimental import pallas as pl
from jax.experimental.pallas import tpu as pltpu
from jax.experimental.pallas import tpu_sc as plsc

N = 10000
E = 320000
D = 128

NC = 2
NS = 16
NW = NC * NS
C = 128
CPT = 80
EP = NW * CPT * C
EPT = CPT * C
NP = N + 48
RPT = 624
TAIL = N - NS * RPT

_mesh = plsc.VectorSubcoreMesh(core_axis_name="c", subcore_axis_name="s")


@functools.partial(
    pl.kernel,
    out_type=jax.ShapeDtypeStruct((NC * N, D), jnp.float32),
    mesh=_mesh,
    scratch_types=(
        [pltpu.VMEM((CPT, C), jnp.int32)]
        + [pltpu.VMEM((C, D), jnp.float32)]
        + [pltpu.SemaphoreType.DMA] * 4
        + [pltpu.VMEM_SHARED((NP, D), jnp.float32)]
    ),
)
def _deg_sc(dst2d_hbm, ones_hbm, zeros_hbm, out_hbm,
            didx, ones_v, s0s, s1s, s2s, s3s, acc):
    c = lax.axis_index("c")
    s = lax.axis_index("s")
    wid = c * NS + s
    ssem = (s0s, s1s, s2s, s3s)

    pltpu.sync_copy(zeros_hbm.at[pl.ds(s * RPT, RPT)], acc.at[pl.ds(s * RPT, RPT)])
    @pl.when(s == 0)
    def _():
        pltpu.sync_copy(zeros_hbm.at[pl.ds(NS * RPT, TAIL)],
                        acc.at[pl.ds(NS * RPT, TAIL)])
    pltpu.sync_copy(dst2d_hbm.at[pl.ds(wid * CPT, CPT)], didx)
    pltpu.sync_copy(ones_hbm, ones_v)
    plsc.subcore_barrier()

    def issue(t, b):
        pltpu.async_copy(ones_v, acc.at[didx.at[t]], ssem[b], add=True)

    def wait(t, b):
        pltpu.make_async_copy(ones_v, acc.at[didx.at[t]], ssem[b]).wait()

    issue(0, 0)
    issue(1, 1)
    issue(2, 2)
    issue(3, 3)

    def body(p, carry):
        t = 4 * p
        wait(t - 4, 0); issue(t, 0)
        wait(t - 3, 1); issue(t + 1, 1)
        wait(t - 2, 2); issue(t + 2, 2)
        wait(t - 1, 3); issue(t + 3, 3)
        return carry

    lax.fori_loop(1, 20, body, 0)
    wait(76, 0)
    wait(77, 1)
    wait(78, 2)
    wait(79, 3)

    plsc.subcore_barrier()
    pltpu.sync_copy(acc.at[pl.ds(s * RPT, RPT)],
                    out_hbm.at[pl.ds(c * N + s * RPT, RPT)])
    @pl.when(s == 0)
    def _():
        pltpu.sync_copy(acc.at[pl.ds(NS * RPT, TAIL)],
                        out_hbm.at[pl.ds(c * N + NS * RPT, TAIL)])


@functools.partial(
    pl.kernel,
    out_type=jax.ShapeDtypeStruct((NC * N, D), jnp.float32),
    mesh=_mesh,
    scratch_types=(
        [pltpu.VMEM((C,), jnp.int32)] * 3
        + [pltpu.VMEM((C,), jnp.int32)] * 3
        + [pltpu.VMEM((C, D), jnp.float32)] * 3
        + [pltpu.SemaphoreType.DMA] * 9
        + [pltpu.VMEM_SHARED((NP, D), jnp.float32)]
    ),
)
def _edge_sc(src_hbm, dst_hbm, g_hbm, zeros_hbm, out_hbm,
             sb0, sb1, sb2, db0, db1, db2, r0, r1, r2,
             g0s, g1s, g2s, i0s, i1s, i2s, d0s, d1s, d2s, acc):
    c = lax.axis_index("c")
    s = lax.axis_index("s")
    wid = c * NS + s
    base = wid * EPT
    rows = (r0, r1, r2)
    sbuf = (sb0, sb1, sb2)
    dbuf = (db0, db1, db2)
    gsem = (g0s, g1s, g2s)
    isem = (i0s, i1s, i2s)
    dsem = (d0s, d1s, d2s)

    pltpu.sync_copy(zeros_hbm.at[pl.ds(s * RPT, RPT)], acc.at[pl.ds(s * RPT, RPT)])
    @pl.when(s == 0)
    def _():
        pltpu.sync_copy(zeros_hbm.at[pl.ds(NS * RPT, TAIL)],
                        acc.at[pl.ds(NS * RPT, TAIL)])
    plsc.subcore_barrier()

    def sidx_pre(t, b):
        return pltpu.async_copy(src_hbm.at[pl.ds(base + t * C, C)],
                                sbuf[b], isem[b])

    def didx_pre(t, b):
        return pltpu.async_copy(dst_hbm.at[pl.ds(base + t * C, C)],
                                dbuf[b], dsem[b])

    def g_issue(t, b):
        pltpu.async_copy(g_hbm.at[sbuf[b]], rows[b], gsem[b])

    def sidx_wait(t, b):
        pltpu.make_async_copy(src_hbm.at[pl.ds(base + t * C, C)],
                              sbuf[b], isem[b]).wait()

    sidx_pre(0, 0)
    sidx_pre(1, 1)
    sidx_pre(2, 2)
    didx_pre(0, 0)
    didx_pre(1, 1)
    didx_pre(2, 2)
    sidx_wait(0, 0)
    g_issue(0, 0)
    sidx_wait(1, 1)
    g_issue(1, 1)

    def visit(t, b, do_g, do_pre):
        b2 = (b + 2) % 3
        pltpu.make_async_copy(g_hbm.at[sbuf[b]], rows[b], gsem[b]).wait()
        if do_g:
            sidx_wait(t + 2, b2)
            g_issue(t + 2, b2)
        if do_pre:
            sidx_pre(t + 3, b)
        pltpu.make_async_copy(dst_hbm.at[pl.ds(base + t * C, C)],
                              dbuf[b], dsem[b]).wait()
        pltpu.sync_copy(rows[b], acc.at[dbuf[b]], add=True)
        if do_pre:
            didx_pre(t + 3, b)

    def body(p, carry):
        t = 3 * p
        visit(t, 0, True, True)
        visit(t + 1, 1, True, True)
        visit(t + 2, 2, True, True)
        return carry

    lax.fori_loop(0, 25, body, 0)
    visit(75, 0, True, True)
    visit(76, 1, True, True)
    visit(77, 2, True, False)
    visit(78, 0, False, False)
    visit(79, 1, False, False)

    plsc.subcore_barrier()
    pltpu.sync_copy(acc.at[pl.ds(s * RPT, RPT)],
                    out_hbm.at[pl.ds(c * N + s * RPT, RPT)])
    @pl.when(s == 0)
    def _():
        pltpu.sync_copy(acc.at[pl.ds(NS * RPT, TAIL)],
                        out_hbm.at[pl.ds(c * N + NS * RPT, TAIL)])


_R = 1000
_NB = N // _R


def _lin_tc_body(x_ref, win_ref, bin_ref, w1_ref, u_ref):
    h0 = jnp.dot(x_ref[...], win_ref[...],
                 preferred_element_type=jnp.float32) + bin_ref[...]
    u_ref[...] = jnp.dot(h0, w1_ref[...], preferred_element_type=jnp.float32)


def _lin_tc(x, w_in, b_in, w1):
    return pl.pallas_call(
        _lin_tc_body,
        grid=(_NB,),
        in_specs=[
            pl.BlockSpec((_R, D), lambda i: (i, 0)),
            pl.BlockSpec((D, D), lambda i: (0, 0)),
            pl.BlockSpec((1, D), lambda i: (0, 0)),
            pl.BlockSpec((D, D), lambda i: (0, 0)),
        ],
        out_specs=pl.BlockSpec((_R, D), lambda i: (i, 0)),
        out_shape=jax.ShapeDtypeStruct((N, D), jnp.float32),
    )(x, w_in, b_in, w1)


def _scale_tc_body(u_ref, ca_ref, cb_ref, dis_ref, g_ref):
    deg = 1.0 + ca_ref[:, 0:1] + cb_ref[:, 0:1]
    dis = lax.rsqrt(deg)
    dis_ref[...] = dis
    g_ref[...] = u_ref[...] * dis


def _scale_tc(u, cnt):
    return pl.pallas_call(
        _scale_tc_body,
        grid=(_NB,),
        in_specs=[
            pl.BlockSpec((_R, D), lambda i: (i, 0)),
            pl.BlockSpec((_R, D), lambda i: (i, 0)),
            pl.BlockSpec((_R, D), lambda i: (_NB + i, 0)),
        ],
        out_specs=[
            pl.BlockSpec((_R, 1), lambda i: (i, 0)),
            pl.BlockSpec((_R, D), lambda i: (i, 0)),
        ],
        out_shape=[
            jax.ShapeDtypeStruct((N, 1), jnp.float32),
            jax.ShapeDtypeStruct((N, D), jnp.float32),
        ],
    )(u, cnt, cnt)


def _mid_tc_body(sa_ref, sb_ref, g_ref, dis_ref, b_ref, w_ref, gn_ref):
    dis = dis_ref[...]
    out = dis * (sa_ref[...] + sb_ref[...] + g_ref[...]) + b_ref[...]
    gn_ref[...] = jnp.dot(out, w_ref[...],
                          preferred_element_type=jnp.float32) * dis


def _mid_tc(s_part, g, dis, b, w):
    return pl.pallas_call(
        _mid_tc_body,
        grid=(_NB,),
        in_specs=[
            pl.BlockSpec((_R, D), lambda i: (i, 0)),
            pl.BlockSpec((_R, D), lambda i: (_NB + i, 0)),
            pl.BlockSpec((_R, D), lambda i: (i, 0)),
            pl.BlockSpec((_R, 1), lambda i: (i, 0)),
            pl.BlockSpec((1, D), lambda i: (0, 0)),
            pl.BlockSpec((D, D), lambda i: (0, 0)),
        ],
        out_specs=pl.BlockSpec((_R, D), lambda i: (i, 0)),
        out_shape=jax.ShapeDtypeStruct((N, D), jnp.float32),
    )(s_part, s_part, g, dis, b, w)


def _final_tc_body(sa_ref, sb_ref, g_ref, dis_ref, b_ref, out_ref):
    out_ref[...] = dis_ref[...] * (sa_ref[...] + sb_ref[...] + g_ref[...]) \
        + b_ref[...]


def _final_tc(s_part, g, dis, b):
    return pl.pallas_call(
        _final_tc_body,
        grid=(_NB,),
        in_specs=[
            pl.BlockSpec((_R, D), lambda i: (i, 0)),
            pl.BlockSpec((_R, D), lambda i: (_NB + i, 0)),
            pl.BlockSpec((_R, D), lambda i: (i, 0)),
            pl.BlockSpec((_R, 1), lambda i: (i, 0)),
            pl.BlockSpec((1, D), lambda i: (0, 0)),
        ],
        out_specs=pl.BlockSpec((_R, D), lambda i: (i, 0)),
        out_shape=jax.ShapeDtypeStruct((N, D), jnp.float32),
    )(s_part, s_part, g, dis, b)


def kernel(x, edge_index, W_in, b_in, W1, b1, W2, b2):
    pad = EP - E
    src_p = jnp.concatenate([edge_index[0], jnp.zeros((pad,), jnp.int32)])
    pad_dst = N + (jnp.arange(pad, dtype=jnp.int32) % (NP - N))
    dst_p = jnp.concatenate([edge_index[1], pad_dst])
    dst2d = dst_p.reshape(-1, C)
    zeros = jnp.zeros((N, D), jnp.float32)
    ones = jnp.ones((C, D), jnp.float32)

    cnt = _deg_sc(dst2d, ones, zeros)
    u = _lin_tc(x, W_in, b_in.reshape(1, D), W1)
    dis, g0 = _scale_tc(u, cnt)

    s0 = _edge_sc(src_p, dst_p, g0, zeros)
    g1 = _mid_tc(s0, g0, dis, b1.reshape(1, D), W2)

    s1 = _edge_sc(src_p, dst_p, g1, zeros)
    return _final_tc(s1, g1, dis, b2.reshape(1, D))


# speedup vs baseline: 3.6168x; 3.0066x over previous
import functools

import jax
import jax.numpy as jnp
from jax import lax
from jax.experimental import pallas as pl
from jax.experimental.pallas import tpu as pltpu
from jax.experimental.pallas import tpu_sc as plsc

N = 10000
E = 320000
D = 128

NC = 2
NS = 16
NW = NC * NS
C = 128
CPT = 80
EP = NW * CPT * C
EPT = CPT * C
NP = N + 48
RPT = 624
TAIL = N - NS * RPT

_mesh = plsc.VectorSubcoreMesh(core_axis_name="c", subcore_axis_name="s")


@functools.partial(
    pl.kernel,
    out_type=jax.ShapeDtypeStruct((NC * N, D), jnp.float32),
    mesh=_mesh,
    scratch_types=(
        [pltpu.VMEM((CPT, C), jnp.int32)]
        + [pltpu.VMEM((C, D), jnp.float32)]
        + [pltpu.SemaphoreType.DMA] * 4
        + [pltpu.VMEM_SHARED((NP, D), jnp.float32)]
    ),
)
def _deg_sc(dst2d_hbm, ones_hbm, zeros_hbm, out_hbm,
            didx, ones_v, s0s, s1s, s2s, s3s, acc):
    c = lax.axis_index("c")
    s = lax.axis_index("s")
    wid = c * NS + s
    ssem = (s0s, s1s, s2s, s3s)

    pltpu.sync_copy(zeros_hbm.at[pl.ds(s * RPT, RPT)], acc.at[pl.ds(s * RPT, RPT)])
    @pl.when(s == 0)
    def _():
        pltpu.sync_copy(zeros_hbm.at[pl.ds(NS * RPT, TAIL)],
                        acc.at[pl.ds(NS * RPT, TAIL)])
    pltpu.sync_copy(dst2d_hbm.at[pl.ds(wid * CPT, CPT)], didx)
    pltpu.sync_copy(ones_hbm, ones_v)
    plsc.subcore_barrier()

    def issue(t, b):
        pltpu.async_copy(ones_v, acc.at[didx.at[t]], ssem[b], add=True)

    def wait(t, b):
        pltpu.make_async_copy(ones_v, acc.at[didx.at[t]], ssem[b]).wait()

    issue(0, 0)
    issue(1, 1)
    issue(2, 2)
    issue(3, 3)

    def body(p, carry):
        t = 4 * p
        wait(t - 4, 0); issue(t, 0)
        wait(t - 3, 1); issue(t + 1, 1)
        wait(t - 2, 2); issue(t + 2, 2)
        wait(t - 1, 3); issue(t + 3, 3)
        return carry

    lax.fori_loop(1, 20, body, 0)
    wait(76, 0)
    wait(77, 1)
    wait(78, 2)
    wait(79, 3)

    plsc.subcore_barrier()
    pltpu.sync_copy(acc.at[pl.ds(s * RPT, RPT)],
                    out_hbm.at[pl.ds(c * N + s * RPT, RPT)])
    @pl.when(s == 0)
    def _():
        pltpu.sync_copy(acc.at[pl.ds(NS * RPT, TAIL)],
                        out_hbm.at[pl.ds(c * N + NS * RPT, TAIL)])


@functools.partial(
    pl.kernel,
    out_type=jax.ShapeDtypeStruct((NC * N, D), jnp.float32),
    mesh=_mesh,
    scratch_types=(
        [pltpu.VMEM((C,), jnp.int32)] * 3
        + [pltpu.VMEM((C,), jnp.int32)] * 3
        + [pltpu.VMEM((C, D), jnp.float32)] * 3
        + [pltpu.SemaphoreType.DMA] * 9
        + [pltpu.VMEM_SHARED((NP, D), jnp.float32)]
    ),
)
def _edge_sc(src_hbm, dst_hbm, g_hbm, zeros_hbm, out_hbm,
             sb0, sb1, sb2, db0, db1, db2, r0, r1, r2,
             g0s, g1s, g2s, i0s, i1s, i2s, d0s, d1s, d2s, acc):
    c = lax.axis_index("c")
    s = lax.axis_index("s")
    wid = c * NS + s
    base = wid * EPT
    rows = (r0, r1, r2)
    sbuf = (sb0, sb1, sb2)
    dbuf = (db0, db1, db2)
    gsem = (g0s, g1s, g2s)
    isem = (i0s, i1s, i2s)
    dsem = (d0s, d1s, d2s)

    pltpu.sync_copy(zeros_hbm.at[pl.ds(s * RPT, RPT)], acc.at[pl.ds(s * RPT, RPT)])
    @pl.when(s == 0)
    def _():
        pltpu.sync_copy(zeros_hbm.at[pl.ds(NS * RPT, TAIL)],
                        acc.at[pl.ds(NS * RPT, TAIL)])
    plsc.subcore_barrier()

    def sidx_pre(t, b):
        return pltpu.async_copy(src_hbm.at[pl.ds(base + t * C, C)],
                                sbuf[b], isem[b])

    def didx_pre(t, b):
        return pltpu.async_copy(dst_hbm.at[pl.ds(base + t * C, C)],
                                dbuf[b], dsem[b])

    def g_issue(t, b):
        pltpu.async_copy(g_hbm.at[sbuf[b]], rows[b], gsem[b])

    def sidx_wait(t, b):
        pltpu.make_async_copy(src_hbm.at[pl.ds(base + t * C, C)],
                              sbuf[b], isem[b]).wait()

    sidx_pre(0, 0)
    sidx_pre(1, 1)
    sidx_pre(2, 2)
    didx_pre(0, 0)
    didx_pre(1, 1)
    didx_pre(2, 2)
    sidx_wait(0, 0)
    g_issue(0, 0)
    sidx_wait(1, 1)
    g_issue(1, 1)

    def visit(t, b, do_g, do_pre):
        b2 = (b + 2) % 3
        pltpu.make_async_copy(g_hbm.at[sbuf[b]], rows[b], gsem[b]).wait()
        if do_g:
            sidx_wait(t + 2, b2)
            g_issue(t + 2, b2)
        if do_pre:
            sidx_pre(t + 3, b)
        pltpu.make_async_copy(dst_hbm.at[pl.ds(base + t * C, C)],
                              dbuf[b], dsem[b]).wait()
        pltpu.sync_copy(rows[b], acc.at[dbuf[b]], add=True)
        if do_pre:
            didx_pre(t + 3, b)

    def body(p, carry):
        t = 3 * p
        visit(t, 0, True, True)
        visit(t + 1, 1, True, True)
        visit(t + 2, 2, True, True)
        return carry

    lax.fori_loop(0, 25, body, 0)
    visit(75, 0, True, True)
    visit(76, 1, True, True)
    visit(77, 2, True, False)
    visit(78, 0, False, False)
    visit(79, 1, False, False)

    plsc.subcore_barrier()
    pltpu.sync_copy(acc.at[pl.ds(s * RPT, RPT)],
                    out_hbm.at[pl.ds(c * N + s * RPT, RPT)])
    @pl.when(s == 0)
    def _():
        pltpu.sync_copy(acc.at[pl.ds(NS * RPT, TAIL)],
                        out_hbm.at[pl.ds(c * N + NS * RPT, TAIL)])


_R = 1000
_NB = N // _R


def _lin_tc_body(x_ref, win_ref, bin_ref, w1_ref, u_ref):
    h0 = jnp.dot(x_ref[...], win_ref[...],
                 preferred_element_type=jnp.float32) + bin_ref[...]
    u_ref[...] = jnp.dot(h0, w1_ref[...], preferred_element_type=jnp.float32)


def _lin_tc(x, w_in, b_in, w1):
    return pl.pallas_call(
        _lin_tc_body,
        grid=(_NB,),
        in_specs=[
            pl.BlockSpec((_R, D), lambda i: (i, 0)),
            pl.BlockSpec((D, D), lambda i: (0, 0)),
            pl.BlockSpec((1, D), lambda i: (0, 0)),
            pl.BlockSpec((D, D), lambda i: (0, 0)),
        ],
        out_specs=pl.BlockSpec((_R, D), lambda i: (i, 0)),
        out_shape=jax.ShapeDtypeStruct((N, D), jnp.float32),
    )(x, w_in, b_in, w1)


def _scale_tc_body(u_ref, ca_ref, cb_ref, dis_ref, g_ref):
    deg = 1.0 + ca_ref[:, 0:1] + cb_ref[:, 0:1]
    dis = lax.rsqrt(deg)
    dis_ref[...] = dis
    g_ref[...] = u_ref[...] * dis


def _scale_tc(u, cnt):
    return pl.pallas_call(
        _scale_tc_body,
        grid=(_NB,),
        in_specs=[
            pl.BlockSpec((_R, D), lambda i: (i, 0)),
            pl.BlockSpec((_R, D), lambda i: (i, 0)),
            pl.BlockSpec((_R, D), lambda i: (_NB + i, 0)),
        ],
        out_specs=[
            pl.BlockSpec((_R, 1), lambda i: (i, 0)),
            pl.BlockSpec((_R, D), lambda i: (i, 0)),
        ],
        out_shape=[
            jax.ShapeDtypeStruct((N, 1), jnp.float32),
            jax.ShapeDtypeStruct((N, D), jnp.float32),
        ],
    )(u, cnt, cnt)


def _mid_tc_body(sa_ref, sb_ref, g_ref, dis_ref, b_ref, w_ref, gn_ref):
    dis = dis_ref[...]
    out = dis * (sa_ref[...] + sb_ref[...] + g_ref[...]) + b_ref[...]
    gn_ref[...] = jnp.dot(out, w_ref[...],
                          preferred_element_type=jnp.float32) * dis


def _mid_tc(s_part, g, dis, b, w):
    return pl.pallas_call(
        _mid_tc_body,
        grid=(_NB,),
        in_specs=[
            pl.BlockSpec((_R, D), lambda i: (i, 0)),
            pl.BlockSpec((_R, D), lambda i: (_NB + i, 0)),
            pl.BlockSpec((_R, D), lambda i: (i, 0)),
            pl.BlockSpec((_R, 1), lambda i: (i, 0)),
            pl.BlockSpec((1, D), lambda i: (0, 0)),
            pl.BlockSpec((D, D), lambda i: (0, 0)),
        ],
        out_specs=pl.BlockSpec((_R, D), lambda i: (i, 0)),
        out_shape=jax.ShapeDtypeStruct((N, D), jnp.float32),
    )(s_part, s_part, g, dis, b, w)


def _final_tc_body(sa_ref, sb_ref, g_ref, dis_ref, b_ref, out_ref):
    out_ref[...] = dis_ref[...] * (sa_ref[...] + sb_ref[...] + g_ref[...]) \
        + b_ref[...]


def _final_tc(s_part, g, dis, b):
    return pl.pallas_call(
        _final_tc_body,
        grid=(_NB,),
        in_specs=[
            pl.BlockSpec((_R, D), lambda i: (i, 0)),
            pl.BlockSpec((_R, D), lambda i: (_NB + i, 0)),
            pl.BlockSpec((_R, D), lambda i: (i, 0)),
            pl.BlockSpec((_R, 1), lambda i: (i, 0)),
            pl.BlockSpec((1, D), lambda i: (0, 0)),
        ],
        out_specs=pl.BlockSpec((_R, D), lambda i: (i, 0)),
        out_shape=jax.ShapeDtypeStruct((N, D), jnp.float32),
    )(s_part, s_part, g, dis, b)


def kernel(x, edge_index, W_in, b_in, W1, b1, W2, b2):
    pad = EP - E
    pad_iota = jnp.arange(pad, dtype=jnp.int32)
    src_p = jnp.concatenate([edge_index[0], pad_iota % N])
    pad_dst = N + (pad_iota % (NP - N))
    dst_p = jnp.concatenate([edge_index[1], pad_dst])
    dst2d = dst_p.reshape(-1, C)
    zeros = jnp.zeros((N, D), jnp.float32)
    ones = jnp.ones((C, D), jnp.float32)

    cnt = _deg_sc(dst2d, ones, zeros)
    u = _lin_tc(x, W_in, b_in.reshape(1, D), W1)
    dis, g0 = _scale_tc(u, cnt)

    s0 = _edge_sc(src_p, dst_p, g0, zeros)
    g1 = _mid_tc(s0, g0, dis, b1.reshape(1, D), W2)

    s1 = _edge_sc(src_p, dst_p, g1, zeros)
    return _final_tc(s1, g1, dis, b2.reshape(1, D))
